# Initial kernel scaffold; baseline (speedup 1.0000x reference)
import jax
import jax.numpy as jnp
from jax.experimental import pallas as pl


def kernel(sents, masks, positions, word_table, pos_table):
    raise NotImplementedError("write your pallas kernel here")


# speedup vs baseline: 1.4163x; 1.4163x over previous
import functools

import jax
import jax.numpy as jnp
from jax import lax
from jax.experimental import pallas as pl
from jax.experimental.pallas import tpu as pltpu
from jax.experimental.pallas import tpu_sc as plsc

B, L = 4096, 50
N = B * L
D_W, D_P = 64, 16
D_OUT = D_W + D_P
NW = 32
PER_W = N // NW
CHUNK = 128
N_CH = PER_W // CHUNK


def _make_kernel():
    mesh = plsc.VectorSubcoreMesh(core_axis_name="c", subcore_axis_name="s")

    @functools.partial(
        pl.kernel,
        mesh=mesh,
        compiler_params=pltpu.CompilerParams(use_tc_tiling_on_sc=False),
        out_type=jax.ShapeDtypeStruct((N, D_OUT), jnp.float32),
        scratch_types=[
            pltpu.VMEM((PER_W,), jnp.int32),
            pltpu.VMEM((PER_W,), jnp.int32),
            pltpu.VMEM((CHUNK, D_W), jnp.float32),
            pltpu.VMEM((CHUNK, D_P), jnp.float32),
            pltpu.SemaphoreType.DMA,
        ],
    )
    def k(idx_hbm, pidx_hbm, wtab_hbm, ptab_hbm, out_hbm,
          idx_v, pidx_v, wbuf, pbuf, sem):
        wid = lax.axis_index("s") * 2 + lax.axis_index("c")
        base = wid * PER_W
        pltpu.sync_copy(idx_hbm.at[pl.ds(base, PER_W)], idx_v)
        pltpu.sync_copy(pidx_hbm.at[pl.ds(base, PER_W)], pidx_v)

        def body(i, _):
            off = i * CHUNK
            cw = pltpu.make_async_copy(
                wtab_hbm.at[idx_v.at[pl.ds(off, CHUNK)]], wbuf, sem)
            cp = pltpu.make_async_copy(
                ptab_hbm.at[pidx_v.at[pl.ds(off, CHUNK)]], pbuf, sem)
            cw.start()
            cp.start()
            cw.wait()
            cp.wait()
            pltpu.sync_copy(
                wbuf, out_hbm.at[pl.ds(base + off, CHUNK), pl.ds(0, D_W)])
            pltpu.sync_copy(
                pbuf, out_hbm.at[pl.ds(base + off, CHUNK), pl.ds(D_W, D_P)])
            return ()

        lax.fori_loop(0, N_CH, body, ())

    return k


_sc_lookup = _make_kernel()


def kernel(sents, masks, positions, word_table, pos_table):
    del masks
    idx = sents.reshape(N).astype(jnp.int32)
    pidx = positions.reshape(N).astype(jnp.int32)
    out = _sc_lookup(idx, pidx, word_table, pos_table)
    return out.reshape(B, L, D_OUT)


# speedup vs baseline: 1.4192x; 1.0021x over previous
import functools

import jax
import jax.numpy as jnp
from jax import lax
from jax.experimental import pallas as pl
from jax.experimental.pallas import tpu as pltpu
from jax.experimental.pallas import tpu_sc as plsc

B, L = 4096, 50
N = B * L
D_W, D_P = 64, 16
D_OUT = D_W + D_P
NW = 32
PER_W = N // NW
CHUNK = 512
SUB = 128

_CHUNKS = []
_off = 0
while _off < PER_W:
    _CHUNKS.append((_off, min(CHUNK, PER_W - _off)))
    _off += CHUNK


def _make_kernel():
    mesh = plsc.VectorSubcoreMesh(core_axis_name="c", subcore_axis_name="s")

    @functools.partial(
        pl.kernel,
        mesh=mesh,
        compiler_params=pltpu.CompilerParams(use_tc_tiling_on_sc=False),
        out_type=jax.ShapeDtypeStruct((N, D_OUT), jnp.float32),
        scratch_types=[
            pltpu.VMEM((PER_W,), jnp.int32),
            pltpu.VMEM((PER_W,), jnp.int32),
            pltpu.VMEM((2, CHUNK, D_W), jnp.float32),
            pltpu.VMEM((2, CHUNK, D_P), jnp.float32),
            pltpu.SemaphoreType.DMA,
            pltpu.SemaphoreType.DMA,
            pltpu.SemaphoreType.DMA,
            pltpu.SemaphoreType.DMA,
        ],
    )
    def k(idx_hbm, pidx_hbm, wtab_hbm, ptab_hbm, out_hbm,
          idx_v, pidx_v, wbuf, pbuf, sg0, sg1, sw0, sw1):
        wid = lax.axis_index("s") * 2 + lax.axis_index("c")
        base = wid * PER_W
        pltpu.sync_copy(idx_hbm.at[pl.ds(base, PER_W)], idx_v)
        pltpu.sync_copy(pidx_hbm.at[pl.ds(base, PER_W)], pidx_v)

        sgs, sws = (sg0, sg1), (sw0, sw1)
        n_ch = len(_CHUNKS)
        gathers = [None] * n_ch
        writes = [None] * n_ch

        def start_gathers(c):
            s = c % 2
            off, ln = _CHUNKS[c]
            cps = []
            so = 0
            while so < ln:
                sl = min(SUB, ln - so)
                cps.append(pltpu.make_async_copy(
                    wtab_hbm.at[idx_v.at[pl.ds(off + so, sl)]],
                    wbuf.at[s, pl.ds(so, sl)], sgs[s]))
                cps.append(pltpu.make_async_copy(
                    ptab_hbm.at[pidx_v.at[pl.ds(off + so, sl)]],
                    pbuf.at[s, pl.ds(so, sl)], sgs[s]))
                so += sl
            for cp in cps:
                cp.start()
            gathers[c] = cps

        def start_writes(c):
            s = c % 2
            off, ln = _CHUNKS[c]
            for cp in gathers[c]:
                cp.wait()
            cps = [
                pltpu.make_async_copy(
                    wbuf.at[s, pl.ds(0, ln)],
                    out_hbm.at[pl.ds(base + off, ln), pl.ds(0, D_W)], sws[s]),
                pltpu.make_async_copy(
                    pbuf.at[s, pl.ds(0, ln)],
                    out_hbm.at[pl.ds(base + off, ln), pl.ds(D_W, D_P)], sws[s]),
            ]
            for cp in cps:
                cp.start()
            writes[c] = cps

        start_gathers(0)
        for c in range(n_ch):
            if c + 1 < n_ch:
                if c >= 1:
                    for cp in writes[c - 1]:
                        cp.wait()
                start_gathers(c + 1)
            start_writes(c)
        for c in (n_ch - 2, n_ch - 1):
            for cp in writes[c]:
                cp.wait()

    return k


_sc_lookup = _make_kernel()


def kernel(sents, masks, positions, word_table, pos_table):
    del masks
    idx = sents.reshape(N).astype(jnp.int32)
    pidx = positions.reshape(N).astype(jnp.int32)
    out = _sc_lookup(idx, pidx, word_table, pos_table)
    return out.reshape(B, L, D_OUT)


# speedup vs baseline: 1.4856x; 1.0468x over previous
import functools

import jax
import jax.numpy as jnp
from jax import lax
from jax.experimental import pallas as pl
from jax.experimental.pallas import tpu as pltpu
from jax.experimental.pallas import tpu_sc as plsc

B, L = 4096, 50
N = B * L
D_W, D_P = 64, 16
D_WP = 128
D_OUT = D_W + D_P
NW = 32
PER_W = N // NW
CHUNK = 256
SUB = 128

_CHUNKS = []
_off = 0
while _off < PER_W:
    _CHUNKS.append((_off, min(CHUNK, PER_W - _off)))
    _off += CHUNK


def _make_kernel():
    mesh = plsc.VectorSubcoreMesh(core_axis_name="c", subcore_axis_name="s")

    @functools.partial(
        pl.kernel,
        mesh=mesh,
        compiler_params=pltpu.CompilerParams(use_tc_tiling_on_sc=False),
        out_type=jax.ShapeDtypeStruct((N, D_OUT), jnp.float32),
        scratch_types=[
            pltpu.VMEM((PER_W,), jnp.int32),
            pltpu.VMEM((PER_W,), jnp.int32),
            pltpu.VMEM((2, CHUNK, D_WP), jnp.float32),
            pltpu.VMEM((2, CHUNK, D_P), jnp.float32),
            pltpu.SemaphoreType.DMA,
            pltpu.SemaphoreType.DMA,
            pltpu.SemaphoreType.DMA,
            pltpu.SemaphoreType.DMA,
        ],
    )
    def k(idx_hbm, pidx_hbm, wtab_hbm, ptab_hbm, out_hbm,
          idx_v, pidx_v, wbuf, pbuf, sg0, sg1, sw0, sw1):
        wid = lax.axis_index("s") * 2 + lax.axis_index("c")
        base = wid * PER_W
        pltpu.sync_copy(idx_hbm.at[pl.ds(base, PER_W)], idx_v)
        pltpu.sync_copy(pidx_hbm.at[pl.ds(base, PER_W)], pidx_v)

        sgs, sws = (sg0, sg1), (sw0, sw1)
        n_ch = len(_CHUNKS)
        gathers = [None] * n_ch
        writes = [None] * n_ch

        def start_gathers(c):
            s = c % 2
            off, ln = _CHUNKS[c]
            cps = []
            so = 0
            while so < ln:
                sl = min(SUB, ln - so)
                cps.append(pltpu.make_async_copy(
                    wtab_hbm.at[idx_v.at[pl.ds(off + so, sl)]],
                    wbuf.at[s, pl.ds(so, sl)], sgs[s]))
                cps.append(pltpu.make_async_copy(
                    ptab_hbm.at[pidx_v.at[pl.ds(off + so, sl)]],
                    pbuf.at[s, pl.ds(so, sl)], sgs[s]))
                so += sl
            for cp in cps:
                cp.start()
            gathers[c] = cps

        def start_writes(c):
            s = c % 2
            off, ln = _CHUNKS[c]
            for cp in gathers[c]:
                cp.wait()
            cps = [
                pltpu.make_async_copy(
                    wbuf.at[s, pl.ds(0, ln), pl.ds(0, D_W)],
                    out_hbm.at[pl.ds(base + off, ln), pl.ds(0, D_W)], sws[s]),
                pltpu.make_async_copy(
                    pbuf.at[s, pl.ds(0, ln)],
                    out_hbm.at[pl.ds(base + off, ln), pl.ds(D_W, D_P)], sws[s]),
            ]
            for cp in cps:
                cp.start()
            writes[c] = cps

        start_gathers(0)
        for c in range(n_ch):
            if c + 1 < n_ch:
                if c >= 1:
                    for cp in writes[c - 1]:
                        cp.wait()
                start_gathers(c + 1)
            start_writes(c)
        for c in (n_ch - 2, n_ch - 1):
            for cp in writes[c]:
                cp.wait()

    return k


_sc_lookup = _make_kernel()


def kernel(sents, masks, positions, word_table, pos_table):
    del masks
    idx = sents.reshape(N).astype(jnp.int32)
    pidx = positions.reshape(N).astype(jnp.int32)
    wtab128 = jnp.pad(word_table, ((0, 0), (0, D_WP - D_W)))
    out = _sc_lookup(idx, pidx, wtab128, pos_table)
    return out.reshape(B, L, D_OUT)


# speedup vs baseline: 1.5308x; 1.0304x over previous
import functools

import jax
import jax.numpy as jnp
from jax import lax
from jax.experimental import pallas as pl
from jax.experimental.pallas import tpu as pltpu
from jax.experimental.pallas import tpu_sc as plsc

B, L = 4096, 50
N = B * L
D_W, D_P = 64, 16
D_WP = 128
D_OUT = D_W + D_P
NW = 32
BB = B // NW
CH_B = 8
CHUNK = CH_B * L
N_CH = BB // CH_B
PER_W = BB * L
SUB = 128


def _make_kernel():
    mesh = plsc.VectorSubcoreMesh(core_axis_name="c", subcore_axis_name="s")

    @functools.partial(
        pl.kernel,
        mesh=mesh,
        compiler_params=pltpu.CompilerParams(use_tc_tiling_on_sc=False),
        out_type=jax.ShapeDtypeStruct((B, L, D_OUT), jnp.float32),
        scratch_types=[
            pltpu.VMEM((PER_W,), jnp.int32),
            pltpu.VMEM((PER_W,), jnp.int32),
            pltpu.VMEM((2, CHUNK, D_W), jnp.float32),
            pltpu.VMEM((2, CHUNK, D_P), jnp.float32),
            pltpu.SemaphoreType.DMA,
            pltpu.SemaphoreType.DMA,
            pltpu.SemaphoreType.DMA,
            pltpu.SemaphoreType.DMA,
        ],
    )
    def k(idx_hbm, pidx_hbm, wtab_hbm, ptab_hbm, out_hbm,
          idx_v, pidx_v, wbuf, pbuf, sg0, sg1, sw0, sw1):
        wid = lax.axis_index("s") * 2 + lax.axis_index("c")
        base = wid * PER_W
        b_base = wid * BB
        pltpu.sync_copy(idx_hbm.at[pl.ds(base, PER_W)], idx_v)
        pltpu.sync_copy(pidx_hbm.at[pl.ds(base, PER_W)], pidx_v)

        sgs, sws = (sg0, sg1), (sw0, sw1)
        gathers = [None] * N_CH

        def start_gathers(c):
            s = c % 2
            off = c * CHUNK
            cps = []
            so = 0
            while so < CHUNK:
                sl = min(SUB, CHUNK - so)
                cps.append(pltpu.make_async_copy(
                    wtab_hbm.at[idx_v.at[pl.ds(off + so, sl)]],
                    wbuf.at[s, pl.ds(so, sl)], sgs[s]))
                cps.append(pltpu.make_async_copy(
                    ptab_hbm.at[pidx_v.at[pl.ds(off + so, sl)]],
                    pbuf.at[s, pl.ds(so, sl)], sgs[s]))
                so += sl
            for cp in cps:
                cp.start()
            gathers[c] = cps

        def write_copies(c):
            s = c % 2
            b0 = b_base + c * CH_B
            cps = []
            for r in range(CH_B):
                cps.append(pltpu.make_async_copy(
                    wbuf.at[s, pl.ds(r * L, L)],
                    out_hbm.at[b0 + r, :, pl.ds(0, D_W)], sws[s]))
                cps.append(pltpu.make_async_copy(
                    pbuf.at[s, pl.ds(r * L, L)],
                    out_hbm.at[b0 + r, :, pl.ds(D_W, D_P)], sws[s]))
            return cps

        def start_writes(c):
            for cp in gathers[c]:
                cp.wait()
            for cp in write_copies(c):
                cp.start()

        start_gathers(0)
        for c in range(N_CH):
            if c + 1 < N_CH:
                if c >= 1:
                    for cp in write_copies(c - 1):
                        cp.wait()
                start_gathers(c + 1)
            start_writes(c)
        for c in (N_CH - 2, N_CH - 1):
            for cp in write_copies(c):
                cp.wait()

    return k


_sc_lookup = _make_kernel()


def kernel(sents, masks, positions, word_table, pos_table):
    del masks
    idx2 = sents.reshape(N).astype(jnp.int32) * 2
    pidx = positions.reshape(N).astype(jnp.int32)
    wtab2m = jnp.pad(word_table, ((0, 0), (0, D_WP - D_W))).reshape(2 * 10**6, D_W)
    return _sc_lookup(idx2, pidx, wtab2m, pos_table)


# speedup vs baseline: 1.7370x; 1.1347x over previous
import functools

import jax
import jax.numpy as jnp
from jax import lax
from jax.experimental import pallas as pl
from jax.experimental.pallas import tpu as pltpu
from jax.experimental.pallas import tpu_sc as plsc

B, L = 4096, 50
N = B * L
D_W, D_P = 64, 16
D_WP = 128
D_OUT = D_W + D_P
NW = 32
BB = B // NW
CH_B = 8
CHUNK = CH_B * L
N_CH = BB // CH_B
PER_W = BB * L
SUB = 128


def _make_kernel():
    mesh = plsc.VectorSubcoreMesh(core_axis_name="c", subcore_axis_name="s")

    @functools.partial(
        pl.kernel,
        mesh=mesh,
        compiler_params=pltpu.CompilerParams(use_tc_tiling_on_sc=False),
        out_type=jax.ShapeDtypeStruct((B, 56, 128), jnp.float32),
        scratch_types=[
            pltpu.VMEM((PER_W,), jnp.int32),
            pltpu.VMEM((PER_W,), jnp.int32),
            pltpu.VMEM((2, CHUNK, D_W), jnp.float32),
            pltpu.VMEM((2, CHUNK, D_P), jnp.float32),
            pltpu.SemaphoreType.DMA,
            pltpu.SemaphoreType.DMA,
            pltpu.SemaphoreType.DMA,
            pltpu.SemaphoreType.DMA,
        ],
    )
    def k(idx_hbm, pidx_hbm, wtab_hbm, ptab_hbm, out_hbm,
          idx_v, pidx_v, wbuf, pbuf, sg0, sg1, sw0, sw1):
        wid = lax.axis_index("s") * 2 + lax.axis_index("c")
        base = wid * PER_W
        b_base = wid * BB
        pltpu.sync_copy(idx_hbm.at[pl.ds(base, PER_W)], idx_v)
        pltpu.sync_copy(pidx_hbm.at[pl.ds(base, PER_W)], pidx_v)

        sgs, sws = (sg0, sg1), (sw0, sw1)
        gathers = [None] * N_CH

        def start_gathers(c):
            s = c % 2
            off = c * CHUNK
            cps = []
            so = 0
            while so < CHUNK:
                sl = min(SUB, CHUNK - so)
                cps.append(pltpu.make_async_copy(
                    wtab_hbm.at[idx_v.at[pl.ds(off + so, sl)]],
                    wbuf.at[s, pl.ds(so, sl)], sgs[s]))
                cps.append(pltpu.make_async_copy(
                    ptab_hbm.at[pidx_v.at[pl.ds(off + so, sl)]],
                    pbuf.at[s, pl.ds(so, sl)], sgs[s]))
                so += sl
            for cp in cps:
                cp.start()
            gathers[c] = cps

        def write_copies(c):
            s = c % 2
            b0 = b_base + c * CH_B
            cps = []
            for r in range(CH_B):
                cps.append(pltpu.make_async_copy(
                    wbuf.at[s, pl.ds(r * L, L)],
                    out_hbm.at[b0 + r, pl.ds(0, L), pl.ds(0, D_W)], sws[s]))
                cps.append(pltpu.make_async_copy(
                    pbuf.at[s, pl.ds(r * L, L)],
                    out_hbm.at[b0 + r, pl.ds(0, L), pl.ds(D_W, D_P)], sws[s]))
            return cps

        def start_writes(c):
            for cp in gathers[c]:
                cp.wait()
            for cp in write_copies(c):
                cp.start()

        start_gathers(0)
        for c in range(N_CH):
            if c + 1 < N_CH:
                if c >= 1:
                    for cp in write_copies(c - 1):
                        cp.wait()
                start_gathers(c + 1)
            start_writes(c)
        for c in (N_CH - 2, N_CH - 1):
            for cp in write_copies(c):
                cp.wait()

    return k


_sc_lookup = _make_kernel()


def kernel(sents, masks, positions, word_table, pos_table):
    del masks
    idx2 = sents.reshape(N).astype(jnp.int32) * 2
    pidx = positions.reshape(N).astype(jnp.int32)
    wtab2m = jnp.pad(word_table, ((0, 0), (0, D_WP - D_W))).reshape(2 * 10**6, D_W)
    out6 = _sc_lookup(idx2, pidx, wtab2m, pos_table)
    return out6[:, :L, :D_OUT]


# speedup vs baseline: 1.7459x; 1.0051x over previous
import functools

import jax
import jax.numpy as jnp
from jax import lax
from jax.experimental import pallas as pl
from jax.experimental.pallas import tpu as pltpu
from jax.experimental.pallas import tpu_sc as plsc

B, L = 4096, 50
N = B * L
D_W, D_P = 64, 16
D_WP = 128
D_OUT = D_W + D_P
NW = 32
BB = B // NW
CH_B = 8
CHUNK = CH_B * L
N_CH = BB // CH_B
PER_W = BB * L
SUB = 400


def _make_kernel():
    mesh = plsc.VectorSubcoreMesh(core_axis_name="c", subcore_axis_name="s")

    @functools.partial(
        pl.kernel,
        mesh=mesh,
        compiler_params=pltpu.CompilerParams(use_tc_tiling_on_sc=False),
        out_type=jax.ShapeDtypeStruct((B, 56, 128), jnp.float32),
        scratch_types=[
            pltpu.VMEM((PER_W,), jnp.int32),
            pltpu.VMEM((PER_W,), jnp.int32),
            pltpu.VMEM((2, CHUNK, D_W), jnp.float32),
            pltpu.VMEM((2, CHUNK, D_P), jnp.float32),
            pltpu.SemaphoreType.DMA,
            pltpu.SemaphoreType.DMA,
            pltpu.SemaphoreType.DMA,
            pltpu.SemaphoreType.DMA,
        ],
    )
    def k(idx_hbm, pidx_hbm, wtab_hbm, ptab_hbm, out_hbm,
          idx_v, pidx_v, wbuf, pbuf, sg0, sg1, sw0, sw1):
        wid = lax.axis_index("s") * 2 + lax.axis_index("c")
        base = wid * PER_W
        b_base = wid * BB
        pltpu.sync_copy(idx_hbm.at[pl.ds(base, PER_W)], idx_v)
        pltpu.sync_copy(pidx_hbm.at[pl.ds(base, PER_W)], pidx_v)

        sgs, sws = (sg0, sg1), (sw0, sw1)
        gathers = [None] * N_CH

        def start_gathers(c):
            s = c % 2
            off = c * CHUNK
            cps = []
            so = 0
            while so < CHUNK:
                sl = min(SUB, CHUNK - so)
                cps.append(pltpu.make_async_copy(
                    wtab_hbm.at[idx_v.at[pl.ds(off + so, sl)]],
                    wbuf.at[s, pl.ds(so, sl)], sgs[s]))
                cps.append(pltpu.make_async_copy(
                    ptab_hbm.at[pidx_v.at[pl.ds(off + so, sl)]],
                    pbuf.at[s, pl.ds(so, sl)], sgs[s]))
                so += sl
            for cp in cps:
                cp.start()
            gathers[c] = cps

        def write_copies(c):
            s = c % 2
            b0 = b_base + c * CH_B
            cps = []
            for r in range(CH_B):
                cps.append(pltpu.make_async_copy(
                    wbuf.at[s, pl.ds(r * L, L)],
                    out_hbm.at[b0 + r, pl.ds(0, L), pl.ds(0, D_W)], sws[s]))
                cps.append(pltpu.make_async_copy(
                    pbuf.at[s, pl.ds(r * L, L)],
                    out_hbm.at[b0 + r, pl.ds(0, L), pl.ds(D_W, D_P)], sws[s]))
            return cps

        def start_writes(c):
            for cp in gathers[c]:
                cp.wait()
            for cp in write_copies(c):
                cp.start()

        start_gathers(0)
        for c in range(N_CH):
            if c + 1 < N_CH:
                if c >= 1:
                    for cp in write_copies(c - 1):
                        cp.wait()
                start_gathers(c + 1)
            start_writes(c)
        for c in (N_CH - 2, N_CH - 1):
            for cp in write_copies(c):
                cp.wait()

    return k


_sc_lookup = _make_kernel()


def kernel(sents, masks, positions, word_table, pos_table):
    del masks
    idx2 = sents.reshape(N).astype(jnp.int32) * 2
    pidx = positions.reshape(N).astype(jnp.int32)
    wtab2m = jnp.pad(word_table, ((0, 0), (0, D_WP - D_W))).reshape(2 * 10**6, D_W)
    out6 = _sc_lookup(idx2, pidx, wtab2m, pos_table)
    return out6[:, :L, :D_OUT]


# speedup vs baseline: 2.1442x; 1.2281x over previous
import functools

import jax
import jax.numpy as jnp
from jax import lax
from jax.experimental import pallas as pl
from jax.experimental.pallas import tpu as pltpu
from jax.experimental.pallas import tpu_sc as plsc

B, L = 4096, 50
N = B * L
D_W, D_P = 64, 16
D_WP = 128
D_OUT = D_W + D_P
NW = 32
BB = B // NW
CH_B = 4
CHUNK = CH_B * L
N_CH = BB // CH_B
NBUF = 4
PER_W = BB * L
SUB = CHUNK


def _make_kernel():
    mesh = plsc.VectorSubcoreMesh(core_axis_name="c", subcore_axis_name="s")

    @functools.partial(
        pl.kernel,
        mesh=mesh,
        compiler_params=pltpu.CompilerParams(use_tc_tiling_on_sc=False),
        out_type=jax.ShapeDtypeStruct((B, 56, 128), jnp.float32),
        scratch_types=[
            pltpu.VMEM((PER_W,), jnp.int32),
            pltpu.VMEM((PER_W,), jnp.int32),
            pltpu.VMEM((NBUF, CHUNK, D_W), jnp.float32),
            pltpu.VMEM((NBUF, CHUNK, D_P), jnp.float32),
        ] + [pltpu.SemaphoreType.DMA] * (2 * NBUF),
    )
    def k(idx_hbm, pidx_hbm, wtab_hbm, ptab_hbm, out_hbm,
          idx_v, pidx_v, wbuf, pbuf, *sems):
        wid = lax.axis_index("s") * 2 + lax.axis_index("c")
        base = wid * PER_W
        b_base = wid * BB
        pltpu.sync_copy(idx_hbm.at[pl.ds(base, PER_W)], idx_v)
        pltpu.sync_copy(pidx_hbm.at[pl.ds(base, PER_W)], pidx_v)

        sgs, sws = sems[:NBUF], sems[NBUF:]
        gathers = [None] * N_CH

        def start_gathers(c):
            s = c % NBUF
            off = c * CHUNK
            cps = []
            so = 0
            while so < CHUNK:
                sl = min(SUB, CHUNK - so)
                cps.append(pltpu.make_async_copy(
                    wtab_hbm.at[idx_v.at[pl.ds(off + so, sl)]],
                    wbuf.at[s, pl.ds(so, sl)], sgs[s]))
                cps.append(pltpu.make_async_copy(
                    ptab_hbm.at[pidx_v.at[pl.ds(off + so, sl)]],
                    pbuf.at[s, pl.ds(so, sl)], sgs[s]))
                so += sl
            for cp in cps:
                cp.start()
            gathers[c] = cps

        def write_copies(c):
            s = c % NBUF
            b0 = b_base + c * CH_B
            cps = []
            for r in range(CH_B):
                cps.append(pltpu.make_async_copy(
                    wbuf.at[s, pl.ds(r * L, L)],
                    out_hbm.at[b0 + r, pl.ds(0, L), pl.ds(0, D_W)], sws[s]))
                cps.append(pltpu.make_async_copy(
                    pbuf.at[s, pl.ds(r * L, L)],
                    out_hbm.at[b0 + r, pl.ds(0, L), pl.ds(D_W, D_P)], sws[s]))
            return cps

        def start_writes(c):
            for cp in gathers[c]:
                cp.wait()
            for cp in write_copies(c):
                cp.start()

        for c in range(NBUF):
            start_gathers(c)
        for c in range(N_CH):
            start_writes(c)
            if c + NBUF < N_CH:
                for cp in write_copies(c):
                    cp.wait()
                start_gathers(c + NBUF)
        for c in range(N_CH - NBUF, N_CH):
            for cp in write_copies(c):
                cp.wait()

    return k


_sc_lookup = _make_kernel()


def kernel(sents, masks, positions, word_table, pos_table):
    del masks
    idx2 = sents.reshape(N).astype(jnp.int32) * 2
    pidx = positions.reshape(N).astype(jnp.int32)
    wtab2m = jnp.pad(word_table, ((0, 0), (0, D_WP - D_W))).reshape(2 * 10**6, D_W)
    out6 = _sc_lookup(idx2, pidx, wtab2m, pos_table)
    return out6[:, :L, :D_OUT]
